# Initial kernel scaffold; baseline (speedup 1.0000x reference)
import jax
import jax.numpy as jnp
from jax.experimental import pallas as pl


def kernel(z, W):
    raise NotImplementedError("write your pallas kernel here")


# speedup vs baseline: 1.2717x; 1.2717x over previous
import functools

import jax
import jax.numpy as jnp
from jax import lax
from jax.experimental import pallas as pl
from jax.experimental.pallas import tpu as pltpu
from jax.experimental.pallas import tpu_sc as plsc

_CB = 8192
_CD = 256
_TOK = 1024
_KCH = 1024
_LOSS_W = 1.25

_NUM_WORKERS = 32
_GROW = 128


def _bf16_rne(x):
    u = lax.bitcast_convert_type(x, jnp.uint32)
    r = ((u + jnp.uint32(0x7FFF) + ((u >> jnp.uint32(16)) & jnp.uint32(1)))
         & jnp.uint32(0xFFFF0000))
    return lax.bitcast_convert_type(r, jnp.float32)


def _argmin_body(z_ref, w_ref, idx_ref, w2_scr, wsq_scr):
    b = pl.program_id(0)

    @pl.when(b == 0)
    def _():
        w = w_ref[...]
        w2_scr[...] = -2.0 * w
        wsq_scr[...] = jnp.sum(w * w, axis=1, keepdims=True)

    zt = z_ref[0]
    zf = jnp.transpose(zt, (1, 0))
    xh = zf[:, :128]
    yh = zf[:, 128:]
    sq = xh * xh + yh * yh
    zsq_col = jnp.sum(sq, axis=1, keepdims=True)
    zsq = jnp.transpose(zsq_col, (1, 0))
    bv = jnp.full((1, _TOK), jnp.inf, jnp.float32)
    bi = jnp.zeros((1, _TOK), jnp.int32)
    windows = ((0, 2736), (2736, 2736), (5472, 2720))
    for w_off, w_len in windows:
        off = 0
        while off < w_len:
            kc = min(_KCH, w_len - off)
            base = w_off + off
            w2c = w2_scr[pl.ds(base, kc), :]
            s2 = lax.dot_general(w2c, zt, (((1,), (0,)), ((), ())),
                                 preferred_element_type=jnp.float32)
            wsqc = wsq_scr[pl.ds(base, kc), :]
            d = (zsq + wsqc) + s2
            cm = jnp.min(d, axis=0, keepdims=True)
            rows = lax.broadcasted_iota(jnp.int32, d.shape, 0) + base
            ci = jnp.min(jnp.where(d == cm, rows, _CB), axis=0, keepdims=True)
            upd = cm < bv
            bi = jnp.where(upd, ci, bi)
            bv = jnp.where(upd, cm, bv)
            off += kc
        bv = _bf16_rne(bv)
    idx_ref[0] = bi


def _compute_indices(z3, W):
    nb = z3.shape[0]
    return pl.pallas_call(
        _argmin_body,
        grid=(nb,),
        in_specs=[
            pl.BlockSpec((1, _CD, _TOK), lambda b: (b, 0, 0)),
            pl.BlockSpec((_CB, _CD), lambda b: (0, 0)),
        ],
        out_specs=pl.BlockSpec((1, 1, _TOK), lambda b: (b, 0, 0)),
        out_shape=jax.ShapeDtypeStruct((nb, 1, _TOK), jnp.int32),
        scratch_shapes=[
            pltpu.VMEM((_CB, _CD), jnp.float32),
            pltpu.VMEM((_CB, 1), jnp.float32),
        ],
    )(z3, W)


def _make_sc_gather(n_rows):
    per_w = n_rows // _NUM_WORKERS
    n_ch = per_w // _GROW
    mesh = plsc.VectorSubcoreMesh(core_axis_name="c", subcore_axis_name="s")

    @functools.partial(
        pl.kernel,
        mesh=mesh,
        out_type=jax.ShapeDtypeStruct((n_rows, _CD), jnp.float32),
        scratch_types=[
            pltpu.VMEM((n_ch, _GROW), jnp.int32),
            pltpu.VMEM((_GROW, _CD), jnp.float32),
            pltpu.VMEM((_GROW, _CD), jnp.float32),
            pltpu.SemaphoreType.DMA,
            pltpu.SemaphoreType.DMA,
        ],
    )
    def gather_k(w_hbm, idx_hbm, out_hbm, idx_v, buf0, buf1, sem0, sem1):
        wid = lax.axis_index("s") * 2 + lax.axis_index("c")
        base = wid * per_w
        pltpu.sync_copy(idx_hbm.at[wid], idx_v)
        bufs = (buf0, buf1)
        sems = (sem0, sem1)
        cps = [pltpu.async_copy(w_hbm.at[idx_v.at[0]], buf0, sem0), None]
        for j in range(n_ch):
            if j + 1 < n_ch:
                cps[(j + 1) % 2] = pltpu.async_copy(
                    w_hbm.at[idx_v.at[j + 1]], bufs[(j + 1) % 2],
                    sems[(j + 1) % 2])
            cps[j % 2].wait()
            pltpu.sync_copy(bufs[j % 2],
                            out_hbm.at[pl.ds(base + j * _GROW, _GROW)])

    return gather_k


def _assemble_body(zq_ref, z_ref, q_ref, loss_ref):
    b = pl.program_id(0)
    nb = pl.num_programs(0)
    zq = zq_ref[0]
    zt = z_ref[0]
    diff = jnp.transpose(zq, (1, 0)) - zt
    q_ref[0] = zt + diff

    @pl.when(b == 0)
    def _():
        loss_ref[...] = jnp.zeros_like(loss_ref)

    loss_ref[...] += jnp.sum(diff * diff, keepdims=True)

    @pl.when(b == nb - 1)
    def _():
        n_el = nb * _CD * _TOK
        loss_ref[...] = loss_ref[...] * (_LOSS_W / n_el)


def _assemble(zq3, z3):
    nb = z3.shape[0]
    return pl.pallas_call(
        _assemble_body,
        grid=(nb,),
        in_specs=[
            pl.BlockSpec((1, _TOK, _CD), lambda b: (b, 0, 0)),
            pl.BlockSpec((1, _CD, _TOK), lambda b: (b, 0, 0)),
        ],
        out_specs=[
            pl.BlockSpec((1, _CD, _TOK), lambda b: (b, 0, 0)),
            pl.BlockSpec((1, 1), lambda b: (0, 0)),
        ],
        out_shape=[
            jax.ShapeDtypeStruct((nb, _CD, _TOK), jnp.float32),
            jax.ShapeDtypeStruct((1, 1), jnp.float32),
        ],
    )(zq3, z3)


def kernel(z, W):
    b, c, h, w = z.shape
    z3 = z.reshape(b, c, h * w)
    idx3 = _compute_indices(z3, W)
    indices = idx3.reshape(b, h * w)
    n_rows = b * h * w
    idx_sc = idx3.reshape(_NUM_WORKERS, n_rows // (_NUM_WORKERS * _GROW),
                          _GROW)
    zq = _make_sc_gather(n_rows)(W, idx_sc)
    q3, loss11 = _assemble(zq.reshape(b, h * w, c), z3)
    quantized = q3.reshape(b, c, h, w)
    return quantized, indices, loss11.reshape(())


# speedup vs baseline: 1.2814x; 1.0076x over previous
import functools

import jax
import jax.numpy as jnp
from jax import lax
from jax.experimental import pallas as pl
from jax.experimental.pallas import tpu as pltpu
from jax.experimental.pallas import tpu_sc as plsc

_CB = 8192
_CD = 256
_TOK = 1024
_KCH = 1024
_LOSS_W = 1.25

_NUM_WORKERS = 32
_GROW = 128


def _bf16_rne(x):
    u = lax.bitcast_convert_type(x, jnp.uint32)
    r = ((u + jnp.uint32(0x7FFF) + ((u >> jnp.uint32(16)) & jnp.uint32(1)))
         & jnp.uint32(0xFFFF0000))
    return lax.bitcast_convert_type(r, jnp.float32)


def _argmin_body(z_ref, zf_ref, w_ref, idx_ref, w2_scr, wsq_scr):
    b = pl.program_id(0)

    @pl.when(b == 0)
    def _():
        w = w_ref[...]
        w2_scr[...] = -2.0 * w
        wsq_scr[...] = jnp.sum(w * w, axis=1, keepdims=True)

    zt = z_ref[0]
    zf = zf_ref[0]
    xh = zf[:, :128]
    yh = zf[:, 128:]
    sq = xh * xh + yh * yh
    zsq_col = jnp.sum(sq, axis=1, keepdims=True)
    zsq = jnp.transpose(zsq_col, (1, 0))
    bv = jnp.full((1, _TOK), jnp.inf, jnp.float32)
    bi = jnp.zeros((1, _TOK), jnp.int32)
    rows_full = lax.broadcasted_iota(jnp.int32, (_KCH, _TOK), 0)
    windows = ((0, 2736), (2736, 2736), (5472, 2720))
    for w_off, w_len in windows:
        off = 0
        while off < w_len:
            kc = min(_KCH, w_len - off)
            base = w_off + off
            w2c = w2_scr[pl.ds(base, kc), :]
            s2 = lax.dot_general(w2c, zt, (((1,), (0,)), ((), ())),
                                 preferred_element_type=jnp.float32)
            wsqc = wsq_scr[pl.ds(base, kc), :]
            d = (zsq + wsqc) + s2
            cm = jnp.min(d, axis=0, keepdims=True)
            rows = rows_full[:kc] if kc < _KCH else rows_full
            ci = jnp.min(jnp.where(d == cm, rows, _CB),
                         axis=0, keepdims=True) + base
            upd = cm < bv
            bi = jnp.where(upd, ci, bi)
            bv = jnp.where(upd, cm, bv)
            off += kc
        bv = _bf16_rne(bv)
    idx_ref[0] = bi


def _compute_indices(z3, zf3, W):
    nb = z3.shape[0]
    return pl.pallas_call(
        _argmin_body,
        grid=(nb,),
        in_specs=[
            pl.BlockSpec((1, _CD, _TOK), lambda b: (b, 0, 0)),
            pl.BlockSpec((1, _TOK, _CD), lambda b: (b, 0, 0)),
            pl.BlockSpec((_CB, _CD), lambda b: (0, 0)),
        ],
        out_specs=pl.BlockSpec((1, 1, _TOK), lambda b: (b, 0, 0)),
        out_shape=jax.ShapeDtypeStruct((nb, 1, _TOK), jnp.int32),
        scratch_shapes=[
            pltpu.VMEM((_CB, _CD), jnp.float32),
            pltpu.VMEM((_CB, 1), jnp.float32),
        ],
    )(z3, zf3, W)


def _make_sc_gather(n_rows):
    per_w = n_rows // _NUM_WORKERS
    n_ch = per_w // _GROW
    mesh = plsc.VectorSubcoreMesh(core_axis_name="c", subcore_axis_name="s")

    @functools.partial(
        pl.kernel,
        mesh=mesh,
        out_type=jax.ShapeDtypeStruct((n_rows, _CD), jnp.float32),
        scratch_types=[
            pltpu.VMEM((n_ch, _GROW), jnp.int32),
            pltpu.VMEM((_GROW, _CD), jnp.float32),
            pltpu.VMEM((_GROW, _CD), jnp.float32),
            pltpu.SemaphoreType.DMA,
            pltpu.SemaphoreType.DMA,
        ],
    )
    def gather_k(w_hbm, idx_hbm, out_hbm, idx_v, buf0, buf1, sem0, sem1):
        wid = lax.axis_index("s") * 2 + lax.axis_index("c")
        base = wid * per_w
        pltpu.sync_copy(idx_hbm.at[wid], idx_v)
        bufs = (buf0, buf1)
        sems = (sem0, sem1)
        cps = [pltpu.async_copy(w_hbm.at[idx_v.at[0]], buf0, sem0), None]
        for j in range(n_ch):
            if j + 1 < n_ch:
                cps[(j + 1) % 2] = pltpu.async_copy(
                    w_hbm.at[idx_v.at[j + 1]], bufs[(j + 1) % 2],
                    sems[(j + 1) % 2])
            cps[j % 2].wait()
            pltpu.sync_copy(bufs[j % 2],
                            out_hbm.at[pl.ds(base + j * _GROW, _GROW)])

    return gather_k


def _assemble_body(zq_ref, z_ref, q_ref, loss_ref):
    b = pl.program_id(0)
    nb = pl.num_programs(0)
    zq = zq_ref[0]
    zt = z_ref[0]
    diff = jnp.transpose(zq, (1, 0)) - zt
    q_ref[0] = zt + diff

    @pl.when(b == 0)
    def _():
        loss_ref[...] = jnp.zeros_like(loss_ref)

    loss_ref[...] += jnp.sum(diff * diff, keepdims=True)

    @pl.when(b == nb - 1)
    def _():
        n_el = nb * _CD * _TOK
        loss_ref[...] = loss_ref[...] * (_LOSS_W / n_el)


def _assemble(zq3, z3):
    nb = z3.shape[0]
    return pl.pallas_call(
        _assemble_body,
        grid=(nb,),
        in_specs=[
            pl.BlockSpec((1, _TOK, _CD), lambda b: (b, 0, 0)),
            pl.BlockSpec((1, _CD, _TOK), lambda b: (b, 0, 0)),
        ],
        out_specs=[
            pl.BlockSpec((1, _CD, _TOK), lambda b: (b, 0, 0)),
            pl.BlockSpec((1, 1), lambda b: (0, 0)),
        ],
        out_shape=[
            jax.ShapeDtypeStruct((nb, _CD, _TOK), jnp.float32),
            jax.ShapeDtypeStruct((1, 1), jnp.float32),
        ],
    )(zq3, z3)


def kernel(z, W):
    b, c, h, w = z.shape
    z3 = z.reshape(b, c, h * w)
    zf3 = jnp.transpose(z3, (0, 2, 1))
    idx3 = _compute_indices(z3, zf3, W)
    indices = idx3.reshape(b, h * w)
    n_rows = b * h * w
    idx_sc = idx3.reshape(_NUM_WORKERS, n_rows // (_NUM_WORKERS * _GROW),
                          _GROW)
    zq = _make_sc_gather(n_rows)(W, idx_sc)
    q3, loss11 = _assemble(zq.reshape(b, h * w, c), z3)
    quantized = q3.reshape(b, c, h, w)
    return quantized, indices, loss11.reshape(())


# speedup vs baseline: 1.2927x; 1.0088x over previous
import functools

import jax
import jax.numpy as jnp
from jax import lax
from jax.experimental import pallas as pl
from jax.experimental.pallas import tpu as pltpu
from jax.experimental.pallas import tpu_sc as plsc

_CB = 8192
_CD = 256
_TOK = 1024
_KCH = 1368
_LOSS_W = 1.25

_NUM_WORKERS = 32
_GROW = 128


def _bf16_rne(x):
    u = lax.bitcast_convert_type(x, jnp.uint32)
    r = ((u + jnp.uint32(0x7FFF) + ((u >> jnp.uint32(16)) & jnp.uint32(1)))
         & jnp.uint32(0xFFFF0000))
    return lax.bitcast_convert_type(r, jnp.float32)


def _argmin_body(z_ref, zf_ref, w_ref, idx_ref, w2_scr, wsq_scr):
    b = pl.program_id(0)

    @pl.when(b == 0)
    def _():
        w = w_ref[...]
        w2_scr[...] = -2.0 * w
        wsq_scr[...] = jnp.sum(w * w, axis=1, keepdims=True)

    zt = z_ref[0]
    zf = zf_ref[0]
    xh = zf[:, :128]
    yh = zf[:, 128:]
    sq = xh * xh + yh * yh
    zsq_col = jnp.sum(sq, axis=1, keepdims=True)
    zsq = jnp.transpose(zsq_col, (1, 0))
    bv = jnp.full((1, _TOK), jnp.inf, jnp.float32)
    bi = jnp.zeros((1, _TOK), jnp.int32)
    rows_full = lax.broadcasted_iota(jnp.int32, (_KCH, _TOK), 0)
    windows = ((0, 2736), (2736, 2736), (5472, 2720))
    for w_off, w_len in windows:
        off = 0
        while off < w_len:
            kc = min(_KCH, w_len - off)
            base = w_off + off
            w2c = w2_scr[pl.ds(base, kc), :]
            s2 = lax.dot_general(w2c, zt, (((1,), (0,)), ((), ())),
                                 preferred_element_type=jnp.float32)
            wsqc = wsq_scr[pl.ds(base, kc), :]
            d = (zsq + wsqc) + s2
            cm = jnp.min(d, axis=0, keepdims=True)
            rows = rows_full[:kc] if kc < _KCH else rows_full
            ci = jnp.min(jnp.where(d == cm, rows, _CB),
                         axis=0, keepdims=True) + base
            upd = cm < bv
            bi = jnp.where(upd, ci, bi)
            bv = jnp.where(upd, cm, bv)
            off += kc
        bv = _bf16_rne(bv)
    idx_ref[0] = bi


def _compute_indices(z3, zf3, W):
    nb = z3.shape[0]
    return pl.pallas_call(
        _argmin_body,
        grid=(nb,),
        in_specs=[
            pl.BlockSpec((1, _CD, _TOK), lambda b: (b, 0, 0)),
            pl.BlockSpec((1, _TOK, _CD), lambda b: (b, 0, 0)),
            pl.BlockSpec((_CB, _CD), lambda b: (0, 0)),
        ],
        out_specs=pl.BlockSpec((1, 1, _TOK), lambda b: (b, 0, 0)),
        out_shape=jax.ShapeDtypeStruct((nb, 1, _TOK), jnp.int32),
        scratch_shapes=[
            pltpu.VMEM((_CB, _CD), jnp.float32),
            pltpu.VMEM((_CB, 1), jnp.float32),
        ],
    )(z3, zf3, W)


def _make_sc_gather(n_rows):
    per_w = n_rows // _NUM_WORKERS
    n_ch = per_w // _GROW
    mesh = plsc.VectorSubcoreMesh(core_axis_name="c", subcore_axis_name="s")

    @functools.partial(
        pl.kernel,
        mesh=mesh,
        out_type=jax.ShapeDtypeStruct((n_rows, _CD), jnp.float32),
        scratch_types=[
            pltpu.VMEM((n_ch, _GROW), jnp.int32),
            pltpu.VMEM((_GROW, _CD), jnp.float32),
            pltpu.VMEM((_GROW, _CD), jnp.float32),
            pltpu.SemaphoreType.DMA,
            pltpu.SemaphoreType.DMA,
        ],
    )
    def gather_k(w_hbm, idx_hbm, out_hbm, idx_v, buf0, buf1, sem0, sem1):
        wid = lax.axis_index("s") * 2 + lax.axis_index("c")
        base = wid * per_w
        pltpu.sync_copy(idx_hbm.at[wid], idx_v)
        bufs = (buf0, buf1)
        sems = (sem0, sem1)
        cps = [pltpu.async_copy(w_hbm.at[idx_v.at[0]], buf0, sem0), None]
        for j in range(n_ch):
            if j + 1 < n_ch:
                cps[(j + 1) % 2] = pltpu.async_copy(
                    w_hbm.at[idx_v.at[j + 1]], bufs[(j + 1) % 2],
                    sems[(j + 1) % 2])
            cps[j % 2].wait()
            pltpu.sync_copy(bufs[j % 2],
                            out_hbm.at[pl.ds(base + j * _GROW, _GROW)])

    return gather_k


def _assemble_body(zq_ref, z_ref, q_ref, loss_ref):
    b = pl.program_id(0)
    nb = pl.num_programs(0)
    zq = zq_ref[0]
    zt = z_ref[0]
    diff = jnp.transpose(zq, (1, 0)) - zt
    q_ref[0] = zt + diff

    @pl.when(b == 0)
    def _():
        loss_ref[...] = jnp.zeros_like(loss_ref)

    loss_ref[...] += jnp.sum(diff * diff, keepdims=True)

    @pl.when(b == nb - 1)
    def _():
        n_el = nb * _CD * _TOK
        loss_ref[...] = loss_ref[...] * (_LOSS_W / n_el)


def _assemble(zq3, z3):
    nb = z3.shape[0]
    return pl.pallas_call(
        _assemble_body,
        grid=(nb,),
        in_specs=[
            pl.BlockSpec((1, _TOK, _CD), lambda b: (b, 0, 0)),
            pl.BlockSpec((1, _CD, _TOK), lambda b: (b, 0, 0)),
        ],
        out_specs=[
            pl.BlockSpec((1, _CD, _TOK), lambda b: (b, 0, 0)),
            pl.BlockSpec((1, 1), lambda b: (0, 0)),
        ],
        out_shape=[
            jax.ShapeDtypeStruct((nb, _CD, _TOK), jnp.float32),
            jax.ShapeDtypeStruct((1, 1), jnp.float32),
        ],
    )(zq3, z3)


def kernel(z, W):
    b, c, h, w = z.shape
    z3 = z.reshape(b, c, h * w)
    zf3 = jnp.transpose(z3, (0, 2, 1))
    idx3 = _compute_indices(z3, zf3, W)
    indices = idx3.reshape(b, h * w)
    n_rows = b * h * w
    idx_sc = idx3.reshape(_NUM_WORKERS, n_rows // (_NUM_WORKERS * _GROW),
                          _GROW)
    zq = _make_sc_gather(n_rows)(W, idx_sc)
    q3, loss11 = _assemble(zq.reshape(b, h * w, c), z3)
    quantized = q3.reshape(b, c, h, w)
    return quantized, indices, loss11.reshape(())
